# Initial kernel scaffold; baseline (speedup 1.0000x reference)
import jax
import jax.numpy as jnp
from jax.experimental import pallas as pl


def kernel(tgt_seq, gauss):
    raise NotImplementedError("write your pallas kernel here")


# speedup vs baseline: 1.4569x; 1.4569x over previous
import functools
import math

import jax
import jax.numpy as jnp
from jax.experimental import pallas as pl
from jax.experimental.pallas import tpu as pltpu

D_MODEL = 256
HALF = D_MODEL // 2
HEIGHT = 20.0
WIDTH = 32.0
N_SPECIAL = 3
PAD_IDX = 0
EOS_IDX = 1
SCALE = 1

_BT = 1024


def _pe_block(tok, gauss):
    mask = jnp.logical_and(tok != PAD_IDX, tok != EOS_IDX)
    a = (tok - N_SPECIAL).astype(jnp.float32)
    yf = jnp.floor(a / (WIDTH / SCALE)) + (SCALE // 2)
    xf = a - (WIDTH / SCALE) * jnp.floor(a / (WIDTH / SCALE)) + (SCALE // 2)
    cx = 2.0 * (xf / WIDTH) - 1.0
    cy = 2.0 * (yf / HEIGHT) - 1.0
    cxb = cx.astype(jnp.bfloat16).astype(jnp.float32)
    cyb = cy.astype(jnp.bfloat16).astype(jnp.float32)
    g0 = gauss[0:1, :].astype(jnp.bfloat16).astype(jnp.float32)
    g1 = gauss[1:2, :].astype(jnp.bfloat16).astype(jnp.float32)
    t = cxb * g0 + cyb * g1
    r = (2.0 * math.pi) * (t - jnp.round(t))
    pe = jnp.concatenate([jnp.sin(r), jnp.cos(r)], axis=-1)
    return jnp.where(mask, pe, 0.0)


def _fused_kernel(tok_ref, gauss_ref, out_ref):
    out_ref[...] = _pe_block(tok_ref[...], gauss_ref[...])


@jax.jit
def kernel(tgt_seq, gauss):
    b, s = tgt_seq.shape
    n = b * s
    tok = tgt_seq.reshape(n, 1)
    grid = (n // _BT,)
    out = pl.pallas_call(
        _fused_kernel,
        grid=grid,
        in_specs=[
            pl.BlockSpec((_BT, 1), lambda i: (i, 0)),
            pl.BlockSpec((2, HALF), lambda i: (0, 0)),
        ],
        out_specs=pl.BlockSpec((_BT, D_MODEL), lambda i: (i, 0)),
        out_shape=jax.ShapeDtypeStruct((n, D_MODEL), jnp.float32),
    )(tok, gauss)
    return out.reshape(b, s, D_MODEL)


# speedup vs baseline: 3.2699x; 2.2445x over previous
import functools
import math

import jax
import jax.numpy as jnp
from jax import lax
from jax.experimental import pallas as pl
from jax.experimental.pallas import tpu as pltpu
from jax.experimental.pallas import tpu_sc as plsc

D_MODEL = 256
HALF = D_MODEL // 2
HEIGHT = 20.0
WIDTH = 32.0
N_SPECIAL = 3

_V_PAD = 648
_NC, _NS = 2, 16
_NW = _NC * _NS
_B = 1024 * 200
_BPW = _B // _NW
_CH = 128
_NCH = _BPW // _CH


def _table_kernel(gauss_ref, tab_ref):
    v = lax.broadcasted_iota(jnp.int32, (_V_PAD, 1), 0)
    valid = v >= 2
    a = (v - N_SPECIAL).astype(jnp.float32)
    q = jnp.floor(a / WIDTH)
    xf = a - WIDTH * q
    cx = 2.0 * (xf / WIDTH) - 1.0
    cy = 2.0 * (q / HEIGHT) - 1.0
    cxb = cx.astype(jnp.bfloat16).astype(jnp.float32)
    cyb = cy.astype(jnp.bfloat16).astype(jnp.float32)
    g0 = gauss_ref[0:1, :].astype(jnp.bfloat16).astype(jnp.float32)
    g1 = gauss_ref[1:2, :].astype(jnp.bfloat16).astype(jnp.float32)
    t = cxb * g0 + cyb * g1
    f = (2.0 * math.pi) * (t - jnp.round(t))
    pe = jnp.concatenate([jnp.sin(f), jnp.cos(f)], axis=-1)
    tab_ref[...] = jnp.where(valid, pe, 0.0)


def _build_table(gauss):
    return pl.pallas_call(
        _table_kernel,
        out_shape=jax.ShapeDtypeStruct((_V_PAD, D_MODEL), jnp.float32),
    )(gauss)


@functools.lru_cache(maxsize=1)
def _make_sc_gather():
    mesh = plsc.VectorSubcoreMesh(core_axis_name="c", subcore_axis_name="s")

    @functools.partial(
        pl.kernel,
        out_type=jax.ShapeDtypeStruct((_B, D_MODEL), jnp.float32),
        mesh=mesh,
        scratch_types=[
            pltpu.VMEM((_NCH, _CH), jnp.int32),
            pltpu.VMEM((_CH, D_MODEL), jnp.float32),
            pltpu.VMEM((_CH, D_MODEL), jnp.float32),
            pltpu.SemaphoreType.DMA,
            pltpu.SemaphoreType.DMA,
        ],
    )
    def _sc_gather(tab_hbm, idx_hbm, out_hbm, idx_v, buf0, buf1, sem0, sem1):
        wid = lax.axis_index("s") * _NC + lax.axis_index("c")
        base = wid * _BPW
        pltpu.sync_copy(idx_hbm.at[wid], idx_v)
        bufs = (buf0, buf1)
        sems = (sem0, sem1)
        pltpu.async_copy(tab_hbm.at[idx_v.at[0]], buf0, sem0)

        def pair(j, _):
            i0 = 2 * j
            for b in range(2):
                i = i0 + b
                nxt = i + 1

                @pl.when(nxt < _NCH)
                def _():
                    pltpu.async_copy(
                        tab_hbm.at[idx_v.at[nxt]], bufs[1 - b], sems[1 - b]
                    )

                pltpu.make_async_copy(
                    tab_hbm.at[idx_v.at[i]], bufs[b], sems[b]
                ).wait()
                pltpu.sync_copy(
                    bufs[b], out_hbm.at[pl.ds(base + i * _CH, _CH)]
                )
            return 0

        lax.fori_loop(0, _NCH // 2, pair, 0)

    return _sc_gather


@jax.jit
def kernel(tgt_seq, gauss):
    b, s = tgt_seq.shape
    table = _build_table(gauss)
    idx = tgt_seq.reshape(_NW, _NCH, _CH)
    out = _make_sc_gather()(table, idx)
    return out.reshape(b, s, D_MODEL)
